# baseline (speedup 1.0000x reference)
import functools

import jax
import jax.numpy as jnp
from jax import lax
from jax.experimental import pallas as pl
from jax.experimental.pallas import tpu as pltpu
from jax.experimental.pallas import tpu_sc as plsc

LANES = 16
NBUF = 4


def kernel(inputs, token_table, pos_table):
    B, T = inputs.shape
    V, E = token_table.shape

    info = plsc.get_sparse_core_info()
    nc, ns = info.num_cores, info.num_subcores
    nw = nc * ns

    seq_per_w = B // nw
    n_groups = seq_per_w // NBUF
    assert B % (nw * NBUF) == 0 and E % LANES == 0 and T % 8 == 0

    mesh = plsc.VectorSubcoreMesh(core_axis_name="c", subcore_axis_name="s")

    @functools.partial(
        pl.kernel,
        mesh=mesh,
        out_type=jax.ShapeDtypeStruct((B, T, E), jnp.float32),
        scratch_types=[
            pltpu.VMEM((seq_per_w, T), jnp.int32),
            [pltpu.VMEM((T, E), jnp.float32) for _ in range(NBUF)],
            pltpu.VMEM((T, E), jnp.float32),
            [pltpu.SemaphoreType.DMA for _ in range(NBUF)],
            [pltpu.SemaphoreType.DMA for _ in range(NBUF)],
        ],
        compiler_params=pltpu.CompilerParams(use_tc_tiling_on_sc=False),
    )
    def run(idx_hbm, tt_hbm, pos_hbm, out_hbm, idx_v, bufs, pos_v,
            sem_g, sem_st):
        wid = lax.axis_index("s") * nc + lax.axis_index("c")
        base = wid * seq_per_w
        pltpu.sync_copy(pos_hbm, pos_v)
        pltpu.sync_copy(idx_hbm.at[pl.ds(base, seq_per_w)], idx_v)

        def start_gather(c, b):
            pltpu.async_copy(tt_hbm.at[idx_v.at[c]], bufs[b], sem_g[b])

        def wait_gather(b):
            pltpu.make_async_copy(
                tt_hbm.at[pl.ds(0, T)], bufs[b], sem_g[b]).wait()

        def start_store(c, b):
            pltpu.async_copy(bufs[b], out_hbm.at[base + c], sem_st[b])

        def wait_store(b):
            pltpu.make_async_copy(
                bufs[b], out_hbm.at[0], sem_st[b]).wait()

        for b in range(NBUF):
            start_gather(b, b)

        def group(g, carry):
            for b in range(NBUF):
                c = g * NBUF + b
                wait_gather(b)

                @plsc.parallel_loop(0, T, unroll=8)
                def addrow(r):
                    for k in range(E // LANES):
                        sl = pl.ds(k * LANES, LANES)
                        bufs[b][r, sl] = bufs[b][r, sl] + pos_v[r, sl]

                start_store(c, b)
                if b == 0:
                    @pl.when(g >= 1)
                    def _():
                        wait_store(NBUF - 1)
                        start_gather(g * NBUF + NBUF - 1, NBUF - 1)
                else:
                    @pl.when(g <= n_groups - 2)
                    def _():
                        wait_store(b - 1)
                        start_gather((g + 1) * NBUF + b - 1, b - 1)
            return carry

        lax.fori_loop(0, n_groups, group, 0)
        for b in range(NBUF):
            wait_store(b)

    return run(inputs.astype(jnp.int32), token_table.astype(jnp.float32),
               pos_table.astype(jnp.float32))
